# Initial kernel scaffold; baseline (speedup 1.0000x reference)
import jax
import jax.numpy as jnp
from jax.experimental import pallas as pl


def kernel(x, edge_weight, W, bias, edge_index):
    raise NotImplementedError("write your pallas kernel here")


# speedup vs baseline: 4.5714x; 4.5714x over previous
import functools

import jax
import jax.numpy as jnp
from jax import lax
from jax.experimental import pallas as pl
from jax.experimental.pallas import tpu as pltpu
from jax.experimental.pallas import tpu_sc as plsc

N = 10000
E = 320000
F = 128
K = 5
LAMBDA_MAX = 2.0

NC = 2
NS = 16
NW = NC * NS
C = 128
E_PAD = ((E + NW * C - 1) // (NW * C)) * (NW * C)
EPW = E_PAD // NW
NCHUNK = EPW // C
N_PAD = 10240
RPT = N_PAD // NS


def _worker_id():
    return lax.axis_index("s") * NC + lax.axis_index("c")


def _deg_body(row_hbm, ew_hbm, out_hbm, idx_v, val_v, zb_v, acc_sh):
    c = lax.axis_index("c")
    s = lax.axis_index("s")
    w = _worker_id()

    def zloop(i, _):
        zb_v[pl.ds(i * 16, 16)] = jnp.zeros((16,), jnp.float32)
        return 0
    lax.fori_loop(0, RPT // 16, zloop, 0)
    pltpu.sync_copy(zb_v, acc_sh.at[pl.ds(s * RPT, RPT)])
    plsc.subcore_barrier()

    def chunk(i, _):
        base = w * EPW + i * C
        pltpu.sync_copy(row_hbm.at[pl.ds(base, C)], idx_v)
        pltpu.sync_copy(ew_hbm.at[pl.ds(base, C)], val_v)
        pltpu.sync_copy(val_v, acc_sh.at[idx_v], add=True)
        return 0
    lax.fori_loop(0, NCHUNK, chunk, 0)
    plsc.subcore_barrier()
    pltpu.sync_copy(acc_sh.at[pl.ds(s * RPT, RPT)], out_hbm.at[c, pl.ds(s * RPT, RPT)])


_deg_call = functools.partial(
    pl.kernel,
    out_type=jax.ShapeDtypeStruct((NC, N_PAD), jnp.float32),
    mesh=plsc.VectorSubcoreMesh(core_axis_name="c", subcore_axis_name="s"),
    compiler_params=pltpu.CompilerParams(needs_layout_passes=False),
    scratch_types=[
        pltpu.VMEM((C,), jnp.int32),
        pltpu.VMEM((C,), jnp.float32),
        pltpu.VMEM((RPT,), jnp.float32),
        pltpu.VMEM_SHARED((N_PAD,), jnp.float32),
    ],
)(_deg_body)


def _lap_body(row_hbm, col_hbm, ew_hbm, dis_hbm, lap_hbm,
              ridx_v, cidx_v, ew_v, lw_v, dis_v):
    w = _worker_id()
    pltpu.sync_copy(dis_hbm, dis_v)

    def chunk(i, _):
        base = w * EPW + i * C
        pltpu.sync_copy(row_hbm.at[pl.ds(base, C)], ridx_v)
        pltpu.sync_copy(col_hbm.at[pl.ds(base, C)], cidx_v)
        pltpu.sync_copy(ew_hbm.at[pl.ds(base, C)], ew_v)
        for j in range(C // 16):
            sl = pl.ds(j * 16, 16)
            dr = plsc.load_gather(dis_v, [ridx_v[sl]])
            dc = plsc.load_gather(dis_v, [cidx_v[sl]])
            lw_v[sl] = (-1.0) * dr * ew_v[sl] * dc
        pltpu.sync_copy(lw_v, lap_hbm.at[pl.ds(base, C)])
        return 0
    lax.fori_loop(0, NCHUNK, chunk, 0)


_lap_call = functools.partial(
    pl.kernel,
    out_type=jax.ShapeDtypeStruct((E_PAD,), jnp.float32),
    mesh=plsc.VectorSubcoreMesh(core_axis_name="c", subcore_axis_name="s"),
    compiler_params=pltpu.CompilerParams(needs_layout_passes=False),
    scratch_types=[
        pltpu.VMEM((C,), jnp.int32),
        pltpu.VMEM((C,), jnp.int32),
        pltpu.VMEM((C,), jnp.float32),
        pltpu.VMEM((C,), jnp.float32),
        pltpu.VMEM((N_PAD,), jnp.float32),
    ],
)(_lap_body)


def _prop_body(h_hbm, row_hbm, col_hbm, lap_hbm, out_hbm,
               ridx_v, cidx_v, lw_v, rows_v, zb_v, sem, acc_sh):
    c = lax.axis_index("c")
    s = lax.axis_index("s")
    w = _worker_id()

    def zloop(i, _):
        for j in range(F // 16):
            zb_v[i, pl.ds(j * 16, 16)] = jnp.zeros((16,), jnp.float32)
        return 0
    lax.fori_loop(0, C, zloop, 0)
    for q in range(RPT // C):
        pltpu.sync_copy(zb_v, acc_sh.at[pl.ds(s * RPT + q * C, C)])
    plsc.subcore_barrier()

    def chunk(i, _):
        base = w * EPW + i * C
        pltpu.sync_copy(row_hbm.at[pl.ds(base, C)], ridx_v)
        pltpu.sync_copy(col_hbm.at[pl.ds(base, C)], cidx_v)
        pltpu.sync_copy(lap_hbm.at[pl.ds(base, C)], lw_v)
        pltpu.async_copy(h_hbm.at[ridx_v], rows_v, sem).wait()

        def scale(g, _):
            lw16 = lw_v[pl.ds(g * 16, 16)]
            for l in range(16):
                e = g * 16 + l
                sv = lw16[l]
                for j in range(F // 16):
                    sl = pl.ds(j * 16, 16)
                    rows_v[e, sl] = rows_v[e, sl] * sv
            return 0
        lax.fori_loop(0, C // 16, scale, 0)
        pltpu.sync_copy(rows_v, acc_sh.at[cidx_v], add=True)
        return 0
    lax.fori_loop(0, NCHUNK, chunk, 0)
    plsc.subcore_barrier()
    pltpu.sync_copy(acc_sh.at[pl.ds(s * RPT, RPT)],
                    out_hbm.at[c, pl.ds(s * RPT, RPT)])


_prop_call = functools.partial(
    pl.kernel,
    out_type=jax.ShapeDtypeStruct((NC, N_PAD, F), jnp.float32),
    mesh=plsc.VectorSubcoreMesh(core_axis_name="c", subcore_axis_name="s"),
    compiler_params=pltpu.CompilerParams(needs_layout_passes=False),
    scratch_types=[
        pltpu.VMEM((C,), jnp.int32),
        pltpu.VMEM((C,), jnp.int32),
        pltpu.VMEM((C,), jnp.float32),
        pltpu.VMEM((C, F), jnp.float32),
        pltpu.VMEM((C, F), jnp.float32),
        pltpu.SemaphoreType.DMA,
        pltpu.VMEM_SHARED((N_PAD, F), jnp.float32),
    ],
)(_prop_body)


def _dis_body(deg_ref, out_ref):
    d = deg_ref[0] + deg_ref[1]
    out_ref[...] = jnp.where(d > 0, 1.0 / jnp.sqrt(d), 0.0)


def _dis_call(deg2):
    return pl.pallas_call(
        _dis_body,
        out_shape=jax.ShapeDtypeStruct((N_PAD // 128, 128), jnp.float32),
    )(deg2)


def _combine_body(a, b, p_ref, prev_ref, out_ref):
    out_ref[...] = a * (p_ref[0] + p_ref[1]) - b * prev_ref[...]


def _combine_call(p, prev, a, b):
    blk = 1024
    grid = N_PAD // blk
    return pl.pallas_call(
        functools.partial(_combine_body, a, b),
        grid=(grid,),
        in_specs=[
            pl.BlockSpec((NC, blk, F), lambda i: (0, i, 0)),
            pl.BlockSpec((blk, F), lambda i: (i, 0)),
        ],
        out_specs=pl.BlockSpec((blk, F), lambda i: (i, 0)),
        out_shape=jax.ShapeDtypeStruct((N_PAD, F), jnp.float32),
    )(p, prev)


def _matmul_body(x_ref, w_ref, b_ref, out_ref):
    out_ref[...] = jnp.dot(
        x_ref[...], w_ref[...], preferred_element_type=jnp.float32,
        precision=lax.Precision.HIGHEST) + b_ref[...]


def _matmul_call(xcat, wr, bias):
    blk = 1024
    grid = N_PAD // blk
    return pl.pallas_call(
        _matmul_body,
        grid=(grid,),
        in_specs=[
            pl.BlockSpec((blk, K * F), lambda i: (i, 0)),
            pl.BlockSpec((K * F, F), lambda i: (0, 0)),
            pl.BlockSpec((1, F), lambda i: (0, 0)),
        ],
        out_specs=pl.BlockSpec((blk, F), lambda i: (i, 0)),
        out_shape=jax.ShapeDtypeStruct((N_PAD, F), jnp.float32),
    )(xcat, wr, bias)


def kernel(x, edge_weight, W, bias, edge_index):
    row = jnp.zeros((E_PAD,), jnp.int32).at[:E].set(edge_index[0])
    col = jnp.zeros((E_PAD,), jnp.int32).at[:E].set(edge_index[1])
    ew = jnp.zeros((E_PAD,), jnp.float32).at[:E].set(edge_weight)
    h0 = jnp.zeros((N_PAD, F), jnp.float32).at[:N].set(x)

    deg2 = _deg_call(row, ew)
    dis = _dis_call(deg2.reshape(NC, N_PAD // 128, 128)).reshape(N_PAD)
    lap = _lap_call(row, col, ew, dis)

    tx = [h0]
    for k in range(1, K):
        p = _prop_call(tx[-1], row, col, lap)
        a, b = (1.0, 0.0) if k == 1 else (2.0, 1.0)
        prev = tx[-1] if k == 1 else tx[-2]
        tx.append(_combine_call(p, prev, a, b))

    xcat = jnp.concatenate(tx, axis=1)
    wr = W.reshape(K * F, F)
    out = _matmul_call(xcat, wr, bias.reshape(1, F))
    return out[:N]
